# Initial kernel scaffold; baseline (speedup 1.0000x reference)
import functools

import jax
import jax.numpy as jnp
from jax import lax
from jax.experimental import pallas as pl
from jax.experimental.pallas import tpu as pltpu
from jax.experimental.pallas import tpu_sc as plsc

_T = 200
_NC = 2
_NS = 16
_NW = _NC * _NS
_CHUNK = 400
_GATHER = 100
_GPC = _CHUNK // _GATHER


def kernel(X, word_table, pos_table):
    B, T = X.shape
    V, D = word_table.shape
    N = B * T
    rows_per_w = N // _NW
    chunks_per_w = rows_per_w // _CHUNK

    x2d = X.reshape(N // _GATHER, _GATHER)

    mesh = plsc.VectorSubcoreMesh(core_axis_name="c", subcore_axis_name="s")

    @functools.partial(
        pl.kernel,
        out_type=jax.ShapeDtypeStruct((N, D), jnp.float32),
        mesh=mesh,
        scratch_types=[
            pltpu.VMEM((_GPC, _GATHER), jnp.int32),
            pltpu.VMEM((_CHUNK, D), jnp.float32),
            pltpu.VMEM((_T, D), jnp.float32),
            pltpu.SemaphoreType.DMA,
        ],
    )
    def emb(x_hbm, tab_hbm, pos_hbm, out_hbm, idx_v, rows_v, pos_v, sem):
        wid = lax.axis_index("s") * _NC + lax.axis_index("c")
        base = wid * rows_per_w
        pltpu.sync_copy(pos_hbm, pos_v)

        def chunk_body(it, carry):
            row0 = base + it * _CHUNK
            pltpu.sync_copy(x_hbm.at[pl.ds(row0 // _GATHER, _GPC)], idx_v)
            cps = [
                pltpu.async_copy(
                    tab_hbm.at[idx_v.at[j]],
                    rows_v.at[pl.ds(j * _GATHER, _GATHER)],
                    sem,
                )
                for j in range(_GPC)
            ]
            for cp in cps:
                cp.wait()

            def add_row(r, c2):
                for s in range(_CHUNK // _T):
                    for c in range(D // 16):
                        sl = pl.ds(c * 16, 16)
                        rows_v[s * _T + r, sl] = (
                            rows_v[s * _T + r, sl] + pos_v[r, sl]
                        )
                return c2

            lax.fori_loop(0, _T, add_row, 0)
            pltpu.sync_copy(rows_v, out_hbm.at[pl.ds(row0, _CHUNK)])
            return carry

        lax.fori_loop(0, chunks_per_w, chunk_body, 0)

    out = emb(x2d, word_table, pos_table)
    return out.reshape(B, T, D)


# speedup vs baseline: 6.8434x; 6.8434x over previous
import functools

import jax
import jax.numpy as jnp
from jax import lax
from jax.experimental import pallas as pl
from jax.experimental.pallas import tpu as pltpu
from jax.experimental.pallas import tpu_sc as plsc

_T = 200
_NC = 2
_NS = 16
_NW = _NC * _NS
_CHUNK = 800
_GATHER = 100
_GPC = _CHUNK // _GATHER


def kernel(X, word_table, pos_table):
    B, T = X.shape
    V, D = word_table.shape
    N = B * T
    rows_per_w = N // _NW
    chunks_per_w = rows_per_w // _CHUNK

    x2d = X.reshape(N // _GATHER, _GATHER)

    mesh = plsc.VectorSubcoreMesh(core_axis_name="c", subcore_axis_name="s")

    @functools.partial(
        pl.kernel,
        out_type=jax.ShapeDtypeStruct((N, D), jnp.float32),
        mesh=mesh,
        scratch_types=[
            pltpu.VMEM((_GPC, _GATHER), jnp.int32),
            pltpu.VMEM((_CHUNK, D), jnp.float32),
            pltpu.VMEM((_T, D), jnp.float32),
            pltpu.SemaphoreType.DMA,
        ],
        compiler_params=pltpu.CompilerParams(use_tc_tiling_on_sc=False),
    )
    def emb(x_hbm, tab_hbm, pos_hbm, out_hbm, idx_v, rows_v, pos_v, sem):
        wid = lax.axis_index("s") * _NC + lax.axis_index("c")
        base = wid * rows_per_w
        pltpu.sync_copy(pos_hbm, pos_v)

        def chunk_body(it, carry):
            row0 = pl.multiple_of(base + it * _CHUNK, _CHUNK)
            irow0 = pl.multiple_of(
                base // _GATHER + it * _GPC, _GPC
            )
            pltpu.sync_copy(x_hbm.at[pl.ds(irow0, _GPC)], idx_v)
            cps = [
                pltpu.async_copy(
                    tab_hbm.at[idx_v.at[j]],
                    rows_v.at[pl.ds(j * _GATHER, _GATHER)],
                    sem,
                )
                for j in range(_GPC)
            ]
            for cp in cps:
                cp.wait()

            def add_row(r, c2):
                for s in range(_CHUNK // _T):
                    for c in range(D // 16):
                        sl = pl.ds(c * 16, 16)
                        rows_v[s * _T + r, sl] = (
                            rows_v[s * _T + r, sl] + pos_v[r, sl]
                        )
                return c2

            lax.fori_loop(0, _T, add_row, 0)
            pltpu.sync_copy(rows_v, out_hbm.at[pl.ds(row0, _CHUNK)])
            return carry

        lax.fori_loop(0, chunks_per_w, chunk_body, 0)

    out = emb(x2d, word_table, pos_table)
    return out.reshape(B, T, D)
